# baseline (speedup 1.0000x reference)
import numpy as np
import jax
import jax.numpy as jnp
from jax.experimental import pallas as pl
from jax.experimental.pallas import tpu as pltpu

_L = 7
_R = _L // 2
_H = 512
_W = 512
_TH = 128
_PW = _W + 2 * _R


def _rings():
    rings = {}
    for ey in range(-_R, _R + 1):
        for ex in range(-_R, _R + 1):
            d2 = ey * ey + ex * ex
            if d2 <= _R * _R:
                rings.setdefault(d2, []).append((ey, ex))
    out = []
    for d2, offs in sorted(rings.items()):
        groups = {}
        for (ey, ex) in offs:
            groups.setdefault(abs(ey), set()).add(ex)
        out.append((d2, sorted((ady, sorted(dxs)) for ady, dxs in groups.items())))
    return out

_RINGS = _rings()


def _scatter_body(le_ref, dk_ref, mask_ref, x_ref, o_ref, pad_ref):
    le = le_ref[pl.program_id(0), 0]
    rad = jnp.abs(x_ref[0, 3]) * le

    @pl.when(pl.program_id(0) == 0)
    def _zero_borders():
        pad_ref[:, 0:_R, :] = jnp.zeros((4, _R, _W + 2 * _R), jnp.float32)
        pad_ref[:, _R + _H:, :] = jnp.zeros((4, _R, _W + 2 * _R), jnp.float32)
        pad_ref[:, :, 0:_R] = jnp.zeros((4, _H + 2 * _R, _R), jnp.float32)
        pad_ref[:, :, _R + _W:] = jnp.zeros((4, _H + 2 * _R, _R), jnp.float32)
    pad_ref[0, _R:_R + _H, _R:_R + _W] = x_ref[0, 0]
    pad_ref[1, _R:_R + _H, _R:_R + _W] = x_ref[0, 1]
    pad_ref[2, _R:_R + _H, _R:_R + _W] = x_ref[0, 2]
    pad_ref[3, _R:_R + _H, _R:_R + _W] = rad

    for y0 in range(0, _H, _TH):
        prad = pad_ref[3, y0:y0 + _TH + 2 * _R, :]
        prgb = [pad_ref[c, y0:y0 + _TH + 2 * _R, :] for c in range(3)]
        acc = [prgb[c][_R:_R + _TH, _R:_R + _W] for c in range(3)]
        acc.append(None)
        for d2, groups in _RINGS:
            if d2 == 0:
                continue
            rey, rex = next((ey, ex) for ey in range(-_R, _R + 1)
                            for ex in range(-_R, _R + 1)
                            if ey * ey + ex * ex == d2)
            d = dk_ref[_R - rey, _R - rex]
            m = mask_ref[_R - rey, _R - rex]
            w_pad = jnp.clip(prad - (d - 1.0), 0.0, 1.0) * m
            p = [w_pad * prgb[c] for c in range(3)]
            p.append(w_pad)
            for ady, dxs in groups:
                for ci in range(4):
                    xs = None
                    for dx in dxs:
                        t = p[ci][:, _R + dx:_R + dx + _W]
                        xs = t if xs is None else xs + t
                    for ey in ({0} if ady == 0 else {-ady, ady}):
                        t = xs[_R + ey:_R + ey + _TH, :]
                        acc[ci] = t if acc[ci] is None else acc[ci] + t

        inv = 1.0 / (acc[3] + (1.0 + 1e-8))
        o_ref[0, 0, y0:y0 + _TH, :] = acc[0] * inv
        o_ref[0, 1, y0:y0 + _TH, :] = acc[1] * inv
        o_ref[0, 2, y0:y0 + _TH, :] = acc[2] * inv


@jax.jit
def kernel(x, lens_effects, diskernel, lens_mask):
    b, c, h, w = x.shape
    out = pl.pallas_call(
        _scatter_body,
        grid=(b,),
        in_specs=[
            pl.BlockSpec((b, 1), lambda i: (0, 0), memory_space=pltpu.SMEM),
            pl.BlockSpec((_L, _L), lambda i: (0, 0), memory_space=pltpu.SMEM),
            pl.BlockSpec((_L, _L), lambda i: (0, 0), memory_space=pltpu.SMEM),
            pl.BlockSpec((1, 4, h, w), lambda i: (i, 0, 0, 0)),
        ],
        out_specs=pl.BlockSpec((1, 3, h, w), lambda i: (i, 0, 0, 0)),
        out_shape=jax.ShapeDtypeStruct((b, 3, h, w), x.dtype),
        scratch_shapes=[pltpu.VMEM((4, h + 2 * _R, w + 2 * _R), jnp.float32)],
    )(lens_effects, diskernel, lens_mask, x)
    return out
